# baseline (speedup 1.0000x reference)
import functools

import jax
import jax.numpy as jnp
from jax import lax
from jax.experimental import pallas as pl
from jax.experimental.pallas import tpu as pltpu
from jax.experimental.pallas import tpu_sc as plsc

B, L, D = 4096, 200, 64
NC, NS = 2, 16
PER_SC = B // NC
RS = 128
PER_TILE_ST = RS // NS
DMAS = PER_SC // RS
PER_TILE_DMA = DMAS // NS if DMAS >= NS else 0

_mesh = plsc.VectorSubcoreMesh(core_axis_name="c", subcore_axis_name="s")


@functools.partial(
    pl.kernel,
    out_type=jax.ShapeDtypeStruct((B, L * D), jnp.float32),
    mesh=_mesh,
    scratch_types=[
        pltpu.VMEM_SHARED((RS, L * D), jnp.float32),
        pltpu.SemaphoreType.DMA,
    ],
)
def _broadcast_table(emb_hbm, out_hbm, shared, sem):
    cid = lax.axis_index("c")
    sid = lax.axis_index("s")
    stages = [
        pltpu.async_copy(emb_hbm, shared.at[sid * PER_TILE_ST + j], sem)
        for j in range(PER_TILE_ST)
    ]
    for s in stages:
        s.wait()
    plsc.subcore_barrier()
    sc_base = cid * PER_SC
    copies = [
        pltpu.async_copy(
            shared,
            out_hbm.at[pl.ds(sc_base + (sid + j * NS) * RS, RS)],
            sem,
        )
        for j in range(PER_TILE_DMA)
    ]
    for c in copies:
        c.wait()


def kernel(x, embedding):
    flat = _broadcast_table(jnp.reshape(embedding, (L * D,)))
    return jnp.reshape(flat, (B, L, D))


# speedup vs baseline: 1.1782x; 1.1782x over previous
import functools

import jax
import jax.numpy as jnp
from jax import lax
from jax.experimental import pallas as pl
from jax.experimental.pallas import tpu as pltpu
from jax.experimental.pallas import tpu_sc as plsc

B, L, D = 4096, 200, 64
NC, NS = 2, 16
NW = NC * NS
PER_W = B // NW
R = 4
CHUNKS = PER_W // R

_mesh = plsc.VectorSubcoreMesh(core_axis_name="c", subcore_axis_name="s")


@functools.partial(
    pl.kernel,
    out_type=jax.ShapeDtypeStruct((B, L * D), jnp.float32),
    mesh=_mesh,
    scratch_types=[
        pltpu.VMEM((R, L * D), jnp.float32),
        pltpu.SemaphoreType.DMA,
    ],
)
def _broadcast_table(emb_hbm, out_hbm, buf, sem):
    wid = lax.axis_index("s") * NC + lax.axis_index("c")
    base = wid * PER_W
    stages = [pltpu.async_copy(emb_hbm, buf.at[r], sem) for r in range(R)]
    for s in stages:
        s.wait()
    copies = [
        pltpu.async_copy(buf, out_hbm.at[pl.ds(base + i * R, R)], sem)
        for i in range(CHUNKS)
    ]
    for c in copies:
        c.wait()


def kernel(x, embedding):
    flat = _broadcast_table(jnp.reshape(embedding, (L * D,)))
    return jnp.reshape(flat, (B, L, D))


# speedup vs baseline: 1.1916x; 1.0114x over previous
import functools

import jax
import jax.numpy as jnp
from jax import lax
from jax.experimental import pallas as pl
from jax.experimental.pallas import tpu as pltpu
from jax.experimental.pallas import tpu_sc as plsc

B, L, D = 4096, 200, 64
NC, NS = 2, 16
NW = NC * NS
PER_W = B // NW
R = 2
CHUNKS = PER_W // R

_mesh = plsc.VectorSubcoreMesh(core_axis_name="c", subcore_axis_name="s")


@functools.partial(
    pl.kernel,
    out_type=jax.ShapeDtypeStruct((B, L * D), jnp.float32),
    mesh=_mesh,
    scratch_types=[
        pltpu.VMEM((R, L * D), jnp.float32),
        pltpu.SemaphoreType.DMA,
    ],
)
def _broadcast_table(emb_hbm, out_hbm, buf, sem):
    wid = lax.axis_index("s") * NC + lax.axis_index("c")
    base = wid * PER_W
    stages = [pltpu.async_copy(emb_hbm, buf.at[r], sem) for r in range(R)]
    for s in stages:
        s.wait()
    copies = [
        pltpu.async_copy(buf, out_hbm.at[pl.ds(base + i * R, R)], sem)
        for i in range(CHUNKS)
    ]
    for c in copies:
        c.wait()


def kernel(x, embedding):
    flat = _broadcast_table(jnp.reshape(embedding, (L * D,)))
    return jnp.reshape(flat, (B, L, D))
